# Initial kernel scaffold; baseline (speedup 1.0000x reference)
import jax
import jax.numpy as jnp
from jax.experimental import pallas as pl


def kernel(x, params):
    raise NotImplementedError("write your pallas kernel here")


# speedup vs baseline: 4.2596x; 4.2596x over previous
import functools

import jax
import jax.numpy as jnp
import numpy as np
from jax import lax
from jax.experimental import pallas as pl
from jax.experimental.pallas import tpu as pltpu
from jax.experimental.pallas import tpu_sc as plsc

KNN = 20
EPS = 1e-5
B = 16
N = 1024
BN = B * N
NW = 32
PT = BN // NW
SENT = np.float32(-3.0e38)
RPAIR = 256
RCONV = 256
CP = 128


def _pair_body(xr_ref, xb_ref, pidx_ref):
    b = pl.program_id(0)
    r = pl.program_id(1)
    xr = xr_ref[...]
    xb = xb_ref[...]
    R = xr.shape[0]
    g = lax.dot_general(xr, xb, (((1,), (1,)), ((), ())),
                        preferred_element_type=jnp.float32)
    inner = -2.0 * g
    xxr = jnp.sum(xr * xr, axis=1, keepdims=True)
    xxb = jnp.sum(xb * xb, axis=1).reshape(1, N)
    p = (-xxr) - inner - xxb
    colv = lax.broadcasted_iota(jnp.int32, (R, N), 1)
    rowv = lax.broadcasted_iota(jnp.int32, (R, N), 0) + r * R
    pf = jnp.where(colv == rowv, SENT, p)
    base = b * N
    selfc = lax.broadcasted_iota(jnp.int32, (R, 1), 0) + (r * R + base)
    pidx_ref[:, 0:1] = selfc
    for j in range(1, KNN):
        mx = jnp.max(pf, axis=1, keepdims=True)
        cand = jnp.where(pf == mx, colv, jnp.int32(2047))
        mi = jnp.min(cand, axis=1, keepdims=True)
        pidx_ref[:, j:j + 1] = mi + base
        pf = jnp.where(cand == mi, SENT, pf)
    pidx_ref[:, KNN:] = jnp.full((R, 32 - KNN), base, jnp.int32)


def _make_pair():
    return pl.pallas_call(
        _pair_body,
        grid=(B, N // RPAIR),
        in_specs=[
            pl.BlockSpec((RPAIR, CP), lambda b, r: (b * (N // RPAIR) + r, 0)),
            pl.BlockSpec((N, CP), lambda b, r: (b, 0)),
        ],
        out_specs=pl.BlockSpec((RPAIR, 32), lambda b, r: (b * (N // RPAIR) + r, 0)),
        out_shape=jax.ShapeDtypeStruct((BN, 32), jnp.int32),
    )


def _make_sc_gather():
    nblk = PT
    mesh = plsc.VectorSubcoreMesh(core_axis_name="c", subcore_axis_name="s")

    def body(pidx_hbm, xp_hbm, fg_hbm, pidx_v, buf0, buf1, sem0, sem1):
        wid = lax.axis_index("s") * 2 + lax.axis_index("c")
        base = wid * PT
        pltpu.sync_copy(pidx_hbm.at[pl.ds(base * 32, PT * 32)], pidx_v)

        bufs = (buf0, buf1)
        sems = (sem0, sem1)

        def issue(q, bufr, sem):
            pltpu.async_copy(
                xp_hbm.at[pidx_v.at[pl.ds(q * 32, KNN)]], bufr, sem)

        issue(0, buf0, sem0)
        issue(1, buf1, sem1)

        def step(i, carry):
            for par in range(2):
                bufr = bufs[par]
                sem = sems[par]
                q = 2 * i + par
                pltpu.make_async_copy(
                    xp_hbm.at[pidx_v.at[pl.ds(0, KNN)]], bufr, sem).wait()
                pltpu.sync_copy(bufr, fg_hbm.at[base + q])

                @pl.when(q + 2 < PT)
                def _():
                    issue(q + 2, bufr, sem)
            return carry

        lax.fori_loop(0, PT // 2, step, 0)

    return pl.kernel(
        body,
        out_type=[jax.ShapeDtypeStruct((BN, KNN, CP), jnp.float32)],
        mesh=mesh,
        scratch_types=[
            pltpu.VMEM((PT * 32,), jnp.int32),
            pltpu.VMEM((KNN, CP), jnp.float32),
            pltpu.VMEM((KNN, CP), jnp.float32),
            pltpu.SemaphoreType.DMA,
            pltpu.SemaphoreType.DMA,
        ],
        compiler_params=pltpu.CompilerParams(needs_layout_passes=False),
    )


def _conv_body(fg_ref, x_ref, w_ref, hm_ref, st_ref):
    fi = x_ref[...]
    w = w_ref[...]
    R = fi.shape[0]
    cout = w.shape[0]
    hm = None
    s1 = jnp.zeros((1, cout), jnp.float32)
    s2 = jnp.zeros((1, cout), jnp.float32)
    for j in range(KNN):
        fj = fg_ref[:, j, :]
        fe = jnp.concatenate([fj - fi, fi], axis=1)
        h = lax.dot_general(fe, w, (((1,), (1,)), ((), ())),
                            preferred_element_type=jnp.float32)
        hm = h if hm is None else jnp.maximum(hm, h)
        s1 = s1 + jnp.sum(h, axis=0, keepdims=True)
        s2 = s2 + jnp.sum(h * h, axis=0, keepdims=True)
    hm_ref[...] = hm
    z = jnp.zeros_like(s1)
    st_ref[...] = jnp.concatenate([s1, s2, z, z, z, z, z, z], axis=0)[None]


def _make_conv(cout):
    nr = N // RCONV
    return pl.pallas_call(
        _conv_body,
        grid=(B, nr),
        in_specs=[
            pl.BlockSpec((RCONV, KNN, CP), lambda b, r: (b * nr + r, 0, 0)),
            pl.BlockSpec((RCONV, CP), lambda b, r: (b * nr + r, 0)),
            pl.BlockSpec((cout, 2 * CP), lambda b, r: (0, 0)),
        ],
        out_specs=[
            pl.BlockSpec((RCONV, cout), lambda b, r: (b * nr + r, 0)),
            pl.BlockSpec((1, 8, cout), lambda b, r: (b * nr + r, 0, 0)),
        ],
        out_shape=[
            jax.ShapeDtypeStruct((BN, cout), jnp.float32),
            jax.ShapeDtypeStruct((B * nr, 8, cout), jnp.float32),
        ],
    )


def _norm_body(m_ref, mean_ref, sv_ref, x_ref):
    v = jnp.maximum((m_ref[...] - mean_ref[...]) / sv_ref[...], 0.0)
    cp = x_ref.shape[1]
    if cp > v.shape[1]:
        v = jnp.concatenate(
            [v, jnp.zeros((v.shape[0], cp - v.shape[1]), jnp.float32)], axis=1)
    x_ref[...] = v


def _make_norm(cout):
    cp = max(cout, CP)
    return pl.pallas_call(
        _norm_body,
        grid=(B,),
        in_specs=[
            pl.BlockSpec((N, cout), lambda b: (b, 0)),
            pl.BlockSpec((1, cout), lambda b: (0, 0)),
            pl.BlockSpec((1, cout), lambda b: (0, 0)),
        ],
        out_specs=pl.BlockSpec((N, cp), lambda b: (b, 0)),
        out_shape=jax.ShapeDtypeStruct((BN, cp), jnp.float32),
    )


def _mm_stats_body(x_ref, w_ref, z_ref, s_ref):
    z = lax.dot_general(x_ref[...], w_ref[...], (((1,), (1,)), ((), ())),
                        preferred_element_type=jnp.float32)
    z_ref[...] = z
    s1 = jnp.sum(z, axis=0, keepdims=True)
    s2 = jnp.sum(z * z, axis=0, keepdims=True)
    s_ref[...] = jnp.concatenate([s1, s2], axis=0)[None]


def _make_mm_stats(cin, cout):
    return pl.pallas_call(
        _mm_stats_body,
        grid=(B,),
        in_specs=[
            pl.BlockSpec((N, cin), lambda b: (b, 0)),
            pl.BlockSpec((cout, cin), lambda b: (0, 0)),
        ],
        out_specs=[
            pl.BlockSpec((N, cout), lambda b: (b, 0)),
            pl.BlockSpec((1, 2, cout), lambda b: (b, 0, 0)),
        ],
        out_shape=[
            jax.ShapeDtypeStruct((BN, cout), jnp.float32),
            jax.ShapeDtypeStruct((B, 2, cout), jnp.float32),
        ],
    )


def _norm_mm_stats_body(z_ref, mean_ref, sv_ref, w_ref, z2_ref, s_ref):
    a = jnp.maximum((z_ref[...] - mean_ref[...]) / sv_ref[...], 0.0)
    z2 = lax.dot_general(a, w_ref[...], (((1,), (1,)), ((), ())),
                         preferred_element_type=jnp.float32)
    z2_ref[...] = z2
    s1 = jnp.sum(z2, axis=0, keepdims=True)
    s2 = jnp.sum(z2 * z2, axis=0, keepdims=True)
    s_ref[...] = jnp.concatenate([s1, s2], axis=0)[None]


def _make_norm_mm_stats(cin, cout):
    return pl.pallas_call(
        _norm_mm_stats_body,
        grid=(B,),
        in_specs=[
            pl.BlockSpec((N, cin), lambda b: (b, 0)),
            pl.BlockSpec((1, cin), lambda b: (0, 0)),
            pl.BlockSpec((1, cin), lambda b: (0, 0)),
            pl.BlockSpec((cout, cin), lambda b: (0, 0)),
        ],
        out_specs=[
            pl.BlockSpec((N, cout), lambda b: (b, 0)),
            pl.BlockSpec((1, 2, cout), lambda b: (b, 0, 0)),
        ],
        out_shape=[
            jax.ShapeDtypeStruct((BN, cout), jnp.float32),
            jax.ShapeDtypeStruct((B, 2, cout), jnp.float32),
        ],
    )


def _norm_mm_bias_body(z_ref, mean_ref, sv_ref, w_ref, b_ref, o_ref):
    a = jnp.maximum((z_ref[...] - mean_ref[...]) / sv_ref[...], 0.0)
    o_ref[...] = lax.dot_general(a, w_ref[...], (((1,), (1,)), ((), ())),
                                 preferred_element_type=jnp.float32) + b_ref[...]


def _make_norm_mm_bias(cin, cout):
    return pl.pallas_call(
        _norm_mm_bias_body,
        grid=(B,),
        in_specs=[
            pl.BlockSpec((N, cin), lambda b: (b, 0)),
            pl.BlockSpec((1, cin), lambda b: (0, 0)),
            pl.BlockSpec((1, cin), lambda b: (0, 0)),
            pl.BlockSpec((cout, cin), lambda b: (0, 0)),
            pl.BlockSpec((1, cout), lambda b: (0, 0)),
        ],
        out_specs=pl.BlockSpec((N, cout), lambda b: (b, 0)),
        out_shape=jax.ShapeDtypeStruct((BN, cout), jnp.float32),
    )


def _edge_layer(xp, w, creal, cout):
    wp = jnp.zeros((cout, 2 * CP), jnp.float32)
    wp = wp.at[:, :creal].set(w[:, :creal])
    wp = wp.at[:, CP:CP + creal].set(w[:, creal:])
    pidx = _make_pair()(xp, xp)
    fg, = _make_sc_gather()(pidx.reshape(-1), xp)
    hm, st = _make_conv(cout)(fg, xp, wp)
    s = jnp.sum(st, axis=0)
    M = float(BN * KNN)
    mean = s[0] / M
    var = s[1] / M - mean * mean
    sv = jnp.sqrt(var + EPS)
    return _make_norm(cout)(hm, mean.reshape(1, cout), sv.reshape(1, cout))


def _head_bn(sp):
    s = jnp.sum(sp, axis=0)
    mean = s[0] / float(BN)
    var = s[1] / float(BN) - mean * mean
    return mean, jnp.sqrt(var + EPS)


def kernel(x, params):
    p = params
    x0 = jnp.pad(x.reshape(BN, 3), ((0, 0), (0, CP - 3)))
    x1 = _edge_layer(x0, p['W1'], 3, 64)
    x2 = _edge_layer(x1, p['W2'], 64, 64)
    x3 = _edge_layer(x2, p['W3'], 64, 128)
    x4 = _edge_layer(x3, p['W4'], 128, 256)
    xc = jnp.concatenate([x1[:, :64], x2[:, :64], x3, x4], axis=1)

    z1, s1p = _make_mm_stats(512, 1024)(xc, p['L1W'])
    mean1, sv1 = _head_bn(s1p)
    z2, s2p = _make_norm_mm_stats(1024, 256)(
        z1, mean1.reshape(1, 1024), sv1.reshape(1, 1024), p['L2W'])
    mean2, sv2 = _head_bn(s2p)
    w3 = jnp.pad(p['L3W'], ((0, 3), (0, 0)))
    b3 = jnp.pad(p['L3b'], (0, 3)).reshape(1, 16)
    out = _make_norm_mm_bias(256, 16)(
        z2, mean2.reshape(1, 256), sv2.reshape(1, 256), w3, b3)
    return out[:, :13].reshape(B, N, 13)
